# Initial kernel scaffold; baseline (speedup 1.0000x reference)
import jax
import jax.numpy as jnp
from jax.experimental import pallas as pl


def kernel(x, W_ih, W_hh, b_ih, b_hh, ln_gamma, ln_beta, W_g, W_s, b_s, W_fc, b_fc, adj_val, src, dst):
    raise NotImplementedError("write your pallas kernel here")


# speedup vs baseline: 3.4082x; 3.4082x over previous
import functools

import jax
import jax.numpy as jnp
from jax import lax
from jax.experimental import pallas as pl
from jax.experimental.pallas import tpu as pltpu
from jax.experimental.pallas import tpu_sc as plsc

N = 50000
E = 800000
B = 2
T = 16
FIN = 8
H = 64
G = 32


BM = 1000
NB = N // BM


def _lstm_body(x_ref, wih_ref, whh_ref, bias_ref, gam_ref, bet_ref,
               wg_ref, ws_ref, bs_ref, xt_ref, sc_ref):
    xb = x_ref[0]
    h = jnp.zeros((BM, H), jnp.float32)
    c = jnp.zeros((BM, H), jnp.float32)
    wih = wih_ref[...]
    whh = whh_ref[...]
    bias = bias_ref[...]
    for t in range(T):
        g_all = (jnp.dot(xb[t], wih, preferred_element_type=jnp.float32)
                 + jnp.dot(h, whh, preferred_element_type=jnp.float32)
                 + bias)
        gi = jax.nn.sigmoid(g_all[:, 0:H])
        gf = jax.nn.sigmoid(g_all[:, H:2 * H])
        gg = jnp.tanh(g_all[:, 2 * H:3 * H])
        go = jax.nn.sigmoid(g_all[:, 3 * H:4 * H])
        c = gf * c + gi * gg
        h = go * jnp.tanh(c)
    mu = jnp.mean(h, axis=1, keepdims=True)
    d = h - mu
    var = jnp.mean(d * d, axis=1, keepdims=True)
    hn = d * lax.rsqrt(var + 1e-5) * gam_ref[...] + bet_ref[...]
    xt_ref[0] = jnp.dot(hn, wg_ref[...], preferred_element_type=jnp.float32)
    sc_ref[0] = jnp.dot(hn, ws_ref[...],
                        preferred_element_type=jnp.float32) + bs_ref[...]


def _stage1(x, wih_t, whh_t, bias, gamma, beta, wg, ws, bs):
    full = lambda s: pl.BlockSpec(s, lambda b, i: tuple(0 for _ in s))
    return pl.pallas_call(
        _lstm_body,
        grid=(B, NB),
        in_specs=[
            pl.BlockSpec((1, T, BM, FIN), lambda b, i: (b, 0, i, 0)),
            full((FIN, 4 * H)),
            full((H, 4 * H)),
            full((1, 4 * H)),
            full((1, H)),
            full((1, H)),
            full((H, G)),
            full((H, G)),
            full((1, G)),
        ],
        out_specs=[
            pl.BlockSpec((1, BM, G), lambda b, i: (b, i, 0)),
            pl.BlockSpec((1, BM, G), lambda b, i: (b, i, 0)),
        ],
        out_shape=[
            jax.ShapeDtypeStruct((B, N, G), jnp.float32),
            jax.ShapeDtypeStruct((B, N, G), jnp.float32),
        ],
    )(x, wih_t, whh_t, bias, gamma, beta, wg, ws, bs)


NS = 16
SUB = 128
SPC = 4
CH = SUB * SPC
NCHUNK = 98
EPT = CH * NCHUNK
EP = EPT * NS
ZR = 1000
NZ = N // ZR


def _sc_body(xr_hbm, src0_hbm, src1_hbm, dst_hbm, zeros_hbm, out_hbm,
             src_v, dst_v, rows_v, acc, sem):
    c = lax.axis_index("c")
    s = lax.axis_index("s")
    for k in range(4):
        j = k * NS + s

        @pl.when(j < NZ)
        def _():
            pltpu.sync_copy(zeros_hbm, acc.at[pl.ds(j * ZR, ZR)])

    plsc.subcore_barrier()

    def chunk_body(chunk, carry):
        row0 = s * (NCHUNK * SPC) + chunk * SPC

        @pl.when(c == 0)
        def _():
            pltpu.sync_copy(src0_hbm.at[pl.ds(row0, SPC)], src_v)

        @pl.when(c == 1)
        def _():
            pltpu.sync_copy(src1_hbm.at[pl.ds(row0, SPC)], src_v)

        pltpu.sync_copy(dst_hbm.at[pl.ds(row0, SPC)], dst_v)

        def sub_body(j, carry2):
            pltpu.async_copy(xr_hbm.at[src_v.at[j]], rows_v.at[j], sem).wait()
            pltpu.sync_copy(rows_v.at[j], acc.at[dst_v.at[j]], add=True)
            return carry2

        lax.fori_loop(0, SPC, sub_body, 0, unroll=False)
        return carry

    lax.fori_loop(0, NCHUNK, chunk_body, 0, unroll=False)
    plsc.subcore_barrier()
    for k in range(4):
        j = k * NS + s

        @pl.when(j < NZ)
        def _():
            pltpu.sync_copy(acc.at[pl.ds(j * ZR, ZR)],
                            out_hbm.at[pl.ds(c * N + j * ZR, ZR)])


def _stage2(xr2, src0, src1, dstm, zeros):
    mesh = plsc.VectorSubcoreMesh(core_axis_name="c", subcore_axis_name="s")
    kfn = functools.partial(
        pl.kernel,
        mesh=mesh,
        out_type=jax.ShapeDtypeStruct((B * N, G), jnp.float32),
        scratch_types=[
            pltpu.VMEM((SPC, SUB), jnp.int32),
            pltpu.VMEM((SPC, SUB), jnp.int32),
            pltpu.VMEM((SPC, SUB, G), jnp.float32),
            pltpu.VMEM_SHARED((N + 8, G), jnp.float32),
            pltpu.SemaphoreType.DMA,
        ],
        compiler_params=pltpu.CompilerParams(use_tc_tiling_on_sc=False),
    )(_sc_body)
    return kfn(xr2, src0, src1, dstm, zeros)


BM3 = 2000


def _readout_body(sc_ref, agg_ref, wt_ref, bfc_ref, out_ref):
    v = jnp.maximum(sc_ref[0] + agg_ref[0], 0.0)
    out_ref[0] = jnp.sum(v * wt_ref[...], axis=1, keepdims=True) + bfc_ref[0, 0]


def _stage3(selfc, agg3, wt, bfc):
    full = lambda s: pl.BlockSpec(s, lambda b, i: tuple(0 for _ in s))
    return pl.pallas_call(
        _readout_body,
        grid=(B, N // BM3),
        in_specs=[
            pl.BlockSpec((1, BM3, G), lambda b, i: (b, i, 0)),
            pl.BlockSpec((1, BM3, G), lambda b, i: (b, i, 0)),
            full((1, G)),
            full((1, 1)),
        ],
        out_specs=pl.BlockSpec((1, BM3, 1), lambda b, i: (b, i, 0)),
        out_shape=jax.ShapeDtypeStruct((B, N, 1), jnp.float32),
    )(selfc, agg3, wt, bfc)


def kernel(x, W_ih, W_hh, b_ih, b_hh, ln_gamma, ln_beta, W_g, W_s, b_s,
           W_fc, b_fc, adj_val, src, dst):
    wih_t = W_ih.T
    whh_t = W_hh.T
    bias = (b_ih + b_hh).reshape(1, 4 * H)
    gamma = ln_gamma.reshape(1, H)
    beta = ln_beta.reshape(1, H)
    bs2 = b_s.reshape(1, G)

    xt_all, selfc = _stage1(x, wih_t, whh_t, bias, gamma, beta, W_g, W_s, bs2)
    xr2 = xt_all.reshape(B * N, G)

    pad = EP - E
    src_p = jnp.concatenate([src, jnp.zeros((pad,), jnp.int32)])
    dst_p = jnp.concatenate([dst, jnp.full((pad,), N, jnp.int32)])
    src0 = src_p.reshape(EP // SUB, SUB)
    src1 = src0 + N
    dstm = dst_p.reshape(EP // SUB, SUB)
    zeros = jnp.zeros((ZR, G), jnp.float32)

    agg2 = _stage2(xr2, src0, src1, dstm, zeros)
    agg3 = agg2.reshape(B, N, G)

    return _stage3(selfc, agg3, W_fc.T.reshape(1, G), b_fc.reshape(1, 1))


# speedup vs baseline: 4.8329x; 1.4180x over previous
import functools

import jax
import jax.numpy as jnp
from jax import lax
from jax.experimental import pallas as pl
from jax.experimental.pallas import tpu as pltpu
from jax.experimental.pallas import tpu_sc as plsc

N = 50000
E = 800000
B = 2
T = 16
FIN = 8
H = 64
G = 32


BMH = 1000
NBH = (N // 2) // BMH


def _sigt(v):
    return 0.5 * jnp.tanh(0.5 * v) + 0.5


def _lstm_body(x_ref, wih_ref, whh_ref, bias_ref, m64_ref, gam_ref, bet_ref,
               wg_ref, ws_ref, bs_ref, xt_ref, sc_ref):
    xb = x_ref[0]
    hp = jnp.zeros((BMH, 2 * H), jnp.float32)
    cp = jnp.zeros((BMH, 2 * H), jnp.float32)
    wih = wih_ref[...]
    whh = whh_ref[...]
    bias = bias_ref[...]
    for t in range(T):
        g_all = (jnp.dot(xb[t], wih, preferred_element_type=jnp.float32)
                 + jnp.dot(hp, whh, preferred_element_type=jnp.float32)
                 + bias)
        gi = _sigt(g_all[:, 0:2 * H])
        gf = _sigt(g_all[:, 2 * H:4 * H])
        gg = jnp.tanh(g_all[:, 4 * H:6 * H])
        go = _sigt(g_all[:, 6 * H:8 * H])
        cp = gf * cp + gi * gg
        hp = go * jnp.tanh(cp)
    mu = jnp.dot(hp, m64_ref[...], preferred_element_type=jnp.float32)
    d = hp - mu
    var = jnp.dot(d * d, m64_ref[...], preferred_element_type=jnp.float32)
    hn = d * lax.rsqrt(var + 1e-5) * gam_ref[...] + bet_ref[...]
    xt_ref[0] = jnp.dot(hn, wg_ref[...], preferred_element_type=jnp.float32)
    sc_ref[0] = jnp.dot(hn, ws_ref[...],
                        preferred_element_type=jnp.float32) + bs_ref[...]


def _stage1(xh, wih2, whh2, bias2, m64, gamma2, beta2, wg2, ws2, bs2):
    full = lambda s: pl.BlockSpec(s, lambda b, i: tuple(0 for _ in s))
    return pl.pallas_call(
        _lstm_body,
        grid=(B, NBH),
        in_specs=[
            pl.BlockSpec((1, T, BMH, 2 * FIN), lambda b, i: (b, 0, i, 0)),
            full((2 * FIN, 8 * H)),
            full((2 * H, 8 * H)),
            full((1, 8 * H)),
            full((2 * H, 2 * H)),
            full((1, 2 * H)),
            full((1, 2 * H)),
            full((2 * H, 2 * G)),
            full((2 * H, 2 * G)),
            full((1, 2 * G)),
        ],
        out_specs=[
            pl.BlockSpec((1, BMH, 2 * G), lambda b, i: (b, i, 0)),
            pl.BlockSpec((1, BMH, 2 * G), lambda b, i: (b, i, 0)),
        ],
        out_shape=[
            jax.ShapeDtypeStruct((B, N // 2, 2 * G), jnp.float32),
            jax.ShapeDtypeStruct((B, N // 2, 2 * G), jnp.float32),
        ],
    )(xh, wih2, whh2, bias2, m64, gamma2, beta2, wg2, ws2, bs2)


def _pack_weights(W_ih, W_hh, b_ih, b_hh, ln_gamma, ln_beta, W_g, W_s, b_s):
    wih_t = W_ih.T
    whh_t = W_hh.T
    bvec = b_ih + b_hh
    wih2 = jnp.zeros((2 * FIN, 8 * H), jnp.float32)
    whh2 = jnp.zeros((2 * H, 8 * H), jnp.float32)
    bias2 = jnp.zeros((8 * H,), jnp.float32)
    for k in range(4):
        wih2 = wih2.at[0:FIN, 2 * k * H:(2 * k + 1) * H].set(
            wih_t[:, k * H:(k + 1) * H])
        wih2 = wih2.at[FIN:2 * FIN, (2 * k + 1) * H:(2 * k + 2) * H].set(
            wih_t[:, k * H:(k + 1) * H])
        whh2 = whh2.at[0:H, 2 * k * H:(2 * k + 1) * H].set(
            whh_t[:, k * H:(k + 1) * H])
        whh2 = whh2.at[H:2 * H, (2 * k + 1) * H:(2 * k + 2) * H].set(
            whh_t[:, k * H:(k + 1) * H])
        bias2 = bias2.at[2 * k * H:(2 * k + 1) * H].set(bvec[k * H:(k + 1) * H])
        bias2 = bias2.at[(2 * k + 1) * H:(2 * k + 2) * H].set(
            bvec[k * H:(k + 1) * H])
    ones64 = jnp.ones((H, H), jnp.float32) / H
    m64 = jnp.zeros((2 * H, 2 * H), jnp.float32)
    m64 = m64.at[0:H, 0:H].set(ones64).at[H:2 * H, H:2 * H].set(ones64)
    gamma2 = jnp.concatenate([ln_gamma, ln_gamma]).reshape(1, 2 * H)
    beta2 = jnp.concatenate([ln_beta, ln_beta]).reshape(1, 2 * H)
    wg2 = jnp.zeros((2 * H, 2 * G), jnp.float32)
    wg2 = wg2.at[0:H, 0:G].set(W_g).at[H:2 * H, G:2 * G].set(W_g)
    ws2 = jnp.zeros((2 * H, 2 * G), jnp.float32)
    ws2 = ws2.at[0:H, 0:G].set(W_s).at[H:2 * H, G:2 * G].set(W_s)
    bs2 = jnp.concatenate([b_s, b_s]).reshape(1, 2 * G)
    return wih2, whh2, bias2.reshape(1, 8 * H), m64, gamma2, beta2, wg2, ws2, bs2


NS = 16
SUB = 128
SPC = 4
CH = SUB * SPC
NCHUNK = 98
EPT = CH * NCHUNK
EP = EPT * NS
ZR = 1000
NZ = N // ZR


def _sc_body(xr_hbm, src0_hbm, src1_hbm, dst_hbm, zeros_hbm, out_hbm,
             src_v, dst_v, rows_v, acc, sem):
    c = lax.axis_index("c")
    s = lax.axis_index("s")
    for k in range(4):
        j = k * NS + s

        @pl.when(j < NZ)
        def _():
            pltpu.sync_copy(zeros_hbm, acc.at[pl.ds(j * ZR, ZR)])

    plsc.subcore_barrier()

    def chunk_body(chunk, carry):
        row0 = s * (NCHUNK * SPC) + chunk * SPC

        @pl.when(c == 0)
        def _():
            pltpu.sync_copy(src0_hbm.at[pl.ds(row0, SPC)], src_v)

        @pl.when(c == 1)
        def _():
            pltpu.sync_copy(src1_hbm.at[pl.ds(row0, SPC)], src_v)

        pltpu.sync_copy(dst_hbm.at[pl.ds(row0, SPC)], dst_v)

        def sub_body(j, carry2):
            pltpu.async_copy(xr_hbm.at[src_v.at[j]], rows_v.at[j], sem).wait()
            pltpu.sync_copy(rows_v.at[j], acc.at[dst_v.at[j]], add=True)
            return carry2

        lax.fori_loop(0, SPC, sub_body, 0, unroll=False)
        return carry

    lax.fori_loop(0, NCHUNK, chunk_body, 0, unroll=False)
    plsc.subcore_barrier()
    for k in range(4):
        j = k * NS + s

        @pl.when(j < NZ)
        def _():
            pltpu.sync_copy(acc.at[pl.ds(j * ZR, ZR)],
                            out_hbm.at[pl.ds(c * N + j * ZR, ZR)])


def _stage2(xr2, src0, src1, dstm, zeros):
    mesh = plsc.VectorSubcoreMesh(core_axis_name="c", subcore_axis_name="s")
    kfn = functools.partial(
        pl.kernel,
        mesh=mesh,
        out_type=jax.ShapeDtypeStruct((B * N, G), jnp.float32),
        scratch_types=[
            pltpu.VMEM((SPC, SUB), jnp.int32),
            pltpu.VMEM((SPC, SUB), jnp.int32),
            pltpu.VMEM((SPC, SUB, G), jnp.float32),
            pltpu.VMEM_SHARED((N + 8, G), jnp.float32),
            pltpu.SemaphoreType.DMA,
        ],
        compiler_params=pltpu.CompilerParams(use_tc_tiling_on_sc=False),
    )(_sc_body)
    return kfn(xr2, src0, src1, dstm, zeros)


BM3 = 2000


def _readout_body(sc_ref, agg_ref, wt_ref, bfc_ref, out_ref):
    v = jnp.maximum(sc_ref[0] + agg_ref[0], 0.0)
    out_ref[0] = jnp.sum(v * wt_ref[...], axis=1, keepdims=True) + bfc_ref[0, 0]


def _stage3(selfc, agg3, wt, bfc):
    full = lambda s: pl.BlockSpec(s, lambda b, i: tuple(0 for _ in s))
    return pl.pallas_call(
        _readout_body,
        grid=(B, N // BM3),
        in_specs=[
            pl.BlockSpec((1, BM3, G), lambda b, i: (b, i, 0)),
            pl.BlockSpec((1, BM3, G), lambda b, i: (b, i, 0)),
            full((1, G)),
            full((1, 1)),
        ],
        out_specs=pl.BlockSpec((1, BM3, 1), lambda b, i: (b, i, 0)),
        out_shape=jax.ShapeDtypeStruct((B, N, 1), jnp.float32),
    )(selfc, agg3, wt, bfc)


def kernel(x, W_ih, W_hh, b_ih, b_hh, ln_gamma, ln_beta, W_g, W_s, b_s,
           W_fc, b_fc, adj_val, src, dst):
    packed = _pack_weights(W_ih, W_hh, b_ih, b_hh, ln_gamma, ln_beta,
                           W_g, W_s, b_s)
    xh = x.reshape(B, T, N // 2, 2 * FIN)
    xt_all, selfc_p = _stage1(xh, *packed)
    selfc = selfc_p.reshape(B, N, G)
    xr2 = xt_all.reshape(B * N, G)

    pad = EP - E
    src_p = jnp.concatenate([src, jnp.zeros((pad,), jnp.int32)])
    dst_p = jnp.concatenate([dst, jnp.full((pad,), N, jnp.int32)])
    src0 = src_p.reshape(EP // SUB, SUB)
    src1 = src0 + N
    dstm = dst_p.reshape(EP // SUB, SUB)
    zeros = jnp.zeros((ZR, G), jnp.float32)

    agg2 = _stage2(xr2, src0, src1, dstm, zeros)
    agg3 = agg2.reshape(B, N, G)

    return _stage3(selfc, agg3, W_fc.T.reshape(1, G), b_fc.reshape(1, 1))


# speedup vs baseline: 5.7355x; 1.1868x over previous
import functools

import jax
import jax.numpy as jnp
from jax import lax
from jax.experimental import pallas as pl
from jax.experimental.pallas import tpu as pltpu
from jax.experimental.pallas import tpu_sc as plsc

N = 50000
E = 800000
B = 2
T = 16
FIN = 8
H = 64
G = 32


BMH = 1000
NBH = (N // 2) // BMH


def _sigt(v):
    return 0.5 * jnp.tanh(0.5 * v) + 0.5


def _lstm_body(x_ref, wih_ref, whh_ref, bias_ref, m64_ref, gam_ref, bet_ref,
               wg_ref, ws_ref, bs_ref, xt_ref, sc_ref):
    xb = x_ref[0]
    hp = jnp.zeros((BMH, 2 * H), jnp.float32)
    cp = jnp.zeros((BMH, 2 * H), jnp.float32)
    wih = wih_ref[...]
    whh = whh_ref[...]
    bias = bias_ref[...]
    for t in range(T):
        g_all = (jnp.dot(xb[t], wih, preferred_element_type=jnp.float32)
                 + jnp.dot(hp, whh, preferred_element_type=jnp.float32)
                 + bias)
        gi = _sigt(g_all[:, 0:2 * H])
        gf = _sigt(g_all[:, 2 * H:4 * H])
        gg = jnp.tanh(g_all[:, 4 * H:6 * H])
        go = _sigt(g_all[:, 6 * H:8 * H])
        cp = gf * cp + gi * gg
        hp = go * jnp.tanh(cp)
    mu = jnp.dot(hp, m64_ref[...], preferred_element_type=jnp.float32)
    d = hp - mu
    var = jnp.dot(d * d, m64_ref[...], preferred_element_type=jnp.float32)
    hn = d * lax.rsqrt(var + 1e-5) * gam_ref[...] + bet_ref[...]
    xt_ref[0] = jnp.dot(hn, wg_ref[...], preferred_element_type=jnp.float32)
    sc_ref[0] = jnp.dot(hn, ws_ref[...],
                        preferred_element_type=jnp.float32) + bs_ref[...]


def _stage1(xh, wih2, whh2, bias2, m64, gamma2, beta2, wg2, ws2, bs2):
    full = lambda s: pl.BlockSpec(s, lambda b, i: tuple(0 for _ in s))
    return pl.pallas_call(
        _lstm_body,
        grid=(B, NBH),
        in_specs=[
            pl.BlockSpec((1, T, BMH, 2 * FIN), lambda b, i: (b, 0, i, 0)),
            full((2 * FIN, 8 * H)),
            full((2 * H, 8 * H)),
            full((1, 8 * H)),
            full((2 * H, 2 * H)),
            full((1, 2 * H)),
            full((1, 2 * H)),
            full((2 * H, 2 * G)),
            full((2 * H, 2 * G)),
            full((1, 2 * G)),
        ],
        out_specs=[
            pl.BlockSpec((1, BMH, 2 * G), lambda b, i: (b, i, 0)),
            pl.BlockSpec((1, BMH, 2 * G), lambda b, i: (b, i, 0)),
        ],
        out_shape=[
            jax.ShapeDtypeStruct((B, N // 2, 2 * G), jnp.float32),
            jax.ShapeDtypeStruct((B, N // 2, 2 * G), jnp.float32),
        ],
    )(xh, wih2, whh2, bias2, m64, gamma2, beta2, wg2, ws2, bs2)


def _pack_weights(W_ih, W_hh, b_ih, b_hh, ln_gamma, ln_beta, W_g, W_s, b_s):
    wih_t = W_ih.T
    whh_t = W_hh.T
    bvec = b_ih + b_hh
    wih2 = jnp.zeros((2 * FIN, 8 * H), jnp.float32)
    whh2 = jnp.zeros((2 * H, 8 * H), jnp.float32)
    bias2 = jnp.zeros((8 * H,), jnp.float32)
    for k in range(4):
        wih2 = wih2.at[0:FIN, 2 * k * H:(2 * k + 1) * H].set(
            wih_t[:, k * H:(k + 1) * H])
        wih2 = wih2.at[FIN:2 * FIN, (2 * k + 1) * H:(2 * k + 2) * H].set(
            wih_t[:, k * H:(k + 1) * H])
        whh2 = whh2.at[0:H, 2 * k * H:(2 * k + 1) * H].set(
            whh_t[:, k * H:(k + 1) * H])
        whh2 = whh2.at[H:2 * H, (2 * k + 1) * H:(2 * k + 2) * H].set(
            whh_t[:, k * H:(k + 1) * H])
        bias2 = bias2.at[2 * k * H:(2 * k + 1) * H].set(bvec[k * H:(k + 1) * H])
        bias2 = bias2.at[(2 * k + 1) * H:(2 * k + 2) * H].set(
            bvec[k * H:(k + 1) * H])
    ones64 = jnp.ones((H, H), jnp.float32) / H
    m64 = jnp.zeros((2 * H, 2 * H), jnp.float32)
    m64 = m64.at[0:H, 0:H].set(ones64).at[H:2 * H, H:2 * H].set(ones64)
    gamma2 = jnp.concatenate([ln_gamma, ln_gamma]).reshape(1, 2 * H)
    beta2 = jnp.concatenate([ln_beta, ln_beta]).reshape(1, 2 * H)
    wg2 = jnp.zeros((2 * H, 2 * G), jnp.float32)
    wg2 = wg2.at[0:H, 0:G].set(W_g).at[H:2 * H, G:2 * G].set(W_g)
    ws2 = jnp.zeros((2 * H, 2 * G), jnp.float32)
    ws2 = ws2.at[0:H, 0:G].set(W_s).at[H:2 * H, G:2 * G].set(W_s)
    bs2 = jnp.concatenate([b_s, b_s]).reshape(1, 2 * G)
    return wih2, whh2, bias2.reshape(1, 8 * H), m64, gamma2, beta2, wg2, ws2, bs2


NS = 16
SUB = 128
SPC = 4
CH = SUB * SPC
NCHUNK = 98
EPT = CH * NCHUNK
EP = EPT * NS
ZR = 1000
NZ = N // ZR


def _sc_body(xr_hbm, src0_hbm, src1_hbm, dst_hbm, zeros_hbm, out_hbm,
             src_v, dst_v, rows_v, acc,
             sem_i, sem_g0, sem_g1, sem_g2, sem_g3):
    c = lax.axis_index("c")
    s = lax.axis_index("s")
    sem_g = [sem_g0, sem_g1, sem_g2, sem_g3]
    for k in range(4):
        j = k * NS + s

        @pl.when(j < NZ)
        def _():
            pltpu.sync_copy(zeros_hbm, acc.at[pl.ds(j * ZR, ZR)])

    plsc.subcore_barrier()

    base = s * (NCHUNK * SPC)

    def idx_start(chunk, par):
        row = base + chunk * SPC

        @pl.when(c == 0)
        def _():
            pltpu.make_async_copy(src0_hbm.at[pl.ds(row, SPC)],
                                  src_v.at[par], sem_i).start()

        @pl.when(c == 1)
        def _():
            pltpu.make_async_copy(src1_hbm.at[pl.ds(row, SPC)],
                                  src_v.at[par], sem_i).start()

        pltpu.make_async_copy(dst_hbm.at[pl.ds(row, SPC)],
                              dst_v.at[par], sem_i).start()

    def idx_wait(chunk, par):
        row = base + chunk * SPC
        pltpu.make_async_copy(src0_hbm.at[pl.ds(row, SPC)],
                              src_v.at[par], sem_i).wait()
        pltpu.make_async_copy(dst_hbm.at[pl.ds(row, SPC)],
                              dst_v.at[par], sem_i).wait()

    def gather(par, j, slot):
        return pltpu.make_async_copy(xr_hbm.at[src_v.at[par, j]],
                                     rows_v.at[slot], sem_g[slot])

    idx_start(0, 0)
    idx_wait(0, 0)
    gather(0, 0, 0).start()

    def chunk_body(chunk, carry):
        par = lax.rem(chunk, 2)
        parn = 1 - par

        @pl.when(chunk < NCHUNK - 1)
        def _():
            idx_start(chunk + 1, parn)

        for j in range(SPC):
            if j < SPC - 1:
                gather(par, j + 1, j + 1).start()
            else:
                @pl.when(chunk < NCHUNK - 1)
                def _():
                    idx_wait(chunk + 1, parn)
                    gather(parn, 0, 0).start()
            gather(par, j, j).wait()
            pltpu.sync_copy(rows_v.at[j], acc.at[dst_v.at[par, j]], add=True)
        return carry

    lax.fori_loop(0, NCHUNK, chunk_body, 0, unroll=False)
    plsc.subcore_barrier()
    for k in range(4):
        j = k * NS + s

        @pl.when(j < NZ)
        def _():
            pltpu.sync_copy(acc.at[pl.ds(j * ZR, ZR)],
                            out_hbm.at[pl.ds(c * N + j * ZR, ZR)])


def _stage2(xr2, src0, src1, dstm, zeros):
    mesh = plsc.VectorSubcoreMesh(core_axis_name="c", subcore_axis_name="s")
    kfn = functools.partial(
        pl.kernel,
        mesh=mesh,
        out_type=jax.ShapeDtypeStruct((B * N, G), jnp.float32),
        scratch_types=[
            pltpu.VMEM((2, SPC, SUB), jnp.int32),
            pltpu.VMEM((2, SPC, SUB), jnp.int32),
            pltpu.VMEM((SPC, SUB, G), jnp.float32),
            pltpu.VMEM_SHARED((N + 8, G), jnp.float32),
            pltpu.SemaphoreType.DMA,
            pltpu.SemaphoreType.DMA,
            pltpu.SemaphoreType.DMA,
            pltpu.SemaphoreType.DMA,
            pltpu.SemaphoreType.DMA,
        ],
        compiler_params=pltpu.CompilerParams(use_tc_tiling_on_sc=False),
    )(_sc_body)
    return kfn(xr2, src0, src1, dstm, zeros)


BM3 = 2000


def _readout_body(sc_ref, agg_ref, wt_ref, bfc_ref, out_ref):
    v = jnp.maximum(sc_ref[0] + agg_ref[0], 0.0)
    out_ref[0] = jnp.sum(v * wt_ref[...], axis=1, keepdims=True) + bfc_ref[0, 0]


def _stage3(selfc, agg3, wt, bfc):
    full = lambda s: pl.BlockSpec(s, lambda b, i: tuple(0 for _ in s))
    return pl.pallas_call(
        _readout_body,
        grid=(B, N // BM3),
        in_specs=[
            pl.BlockSpec((1, BM3, G), lambda b, i: (b, i, 0)),
            pl.BlockSpec((1, BM3, G), lambda b, i: (b, i, 0)),
            full((1, G)),
            full((1, 1)),
        ],
        out_specs=pl.BlockSpec((1, BM3, 1), lambda b, i: (b, i, 0)),
        out_shape=jax.ShapeDtypeStruct((B, N, 1), jnp.float32),
    )(selfc, agg3, wt, bfc)


def kernel(x, W_ih, W_hh, b_ih, b_hh, ln_gamma, ln_beta, W_g, W_s, b_s,
           W_fc, b_fc, adj_val, src, dst):
    packed = _pack_weights(W_ih, W_hh, b_ih, b_hh, ln_gamma, ln_beta,
                           W_g, W_s, b_s)
    xh = x.reshape(B, T, N // 2, 2 * FIN)
    xt_all, selfc_p = _stage1(xh, *packed)
    selfc = selfc_p.reshape(B, N, G)
    xr2 = xt_all.reshape(B * N, G)

    pad = EP - E
    src_p = jnp.concatenate([src, jnp.zeros((pad,), jnp.int32)])
    dst_p = jnp.concatenate([dst, jnp.full((pad,), N, jnp.int32)])
    src0 = src_p.reshape(EP // SUB, SUB)
    src1 = src0 + N
    dstm = dst_p.reshape(EP // SUB, SUB)
    zeros = jnp.zeros((ZR, G), jnp.float32)

    agg2 = _stage2(xr2, src0, src1, dstm, zeros)
    agg3 = agg2.reshape(B, N, G)

    return _stage3(selfc, agg3, W_fc.T.reshape(1, G), b_fc.reshape(1, 1))


# speedup vs baseline: 5.8272x; 1.0160x over previous
import functools

import jax
import jax.numpy as jnp
from jax import lax
from jax.experimental import pallas as pl
from jax.experimental.pallas import tpu as pltpu
from jax.experimental.pallas import tpu_sc as plsc

N = 50000
E = 800000
B = 2
T = 16
FIN = 8
H = 64
G = 32


BMH = 1000
NBH = (N // 2) // BMH


def _sigt(v):
    return 0.5 * jnp.tanh(0.5 * v) + 0.5


def _lstm_body(x_ref, wih_ref, whh_ref, bias_ref, m64_ref, gam_ref, bet_ref,
               wg_ref, ws_ref, bs_ref, xt_ref, sc_ref):
    xb = x_ref[0]
    hp = jnp.zeros((BMH, 2 * H), jnp.float32)
    cp = jnp.zeros((BMH, 2 * H), jnp.float32)
    wih = wih_ref[...]
    whh = whh_ref[...]
    bias = bias_ref[...]
    for t in range(T):
        g_all = (jnp.dot(xb[t], wih, preferred_element_type=jnp.float32)
                 + jnp.dot(hp, whh, preferred_element_type=jnp.float32)
                 + bias)
        gi = _sigt(g_all[:, 0:2 * H])
        gf = _sigt(g_all[:, 2 * H:4 * H])
        gg = jnp.tanh(g_all[:, 4 * H:6 * H])
        go = _sigt(g_all[:, 6 * H:8 * H])
        cp = gf * cp + gi * gg
        hp = go * jnp.tanh(cp)
    mu = jnp.dot(hp, m64_ref[...], preferred_element_type=jnp.float32)
    d = hp - mu
    var = jnp.dot(d * d, m64_ref[...], preferred_element_type=jnp.float32)
    hn = d * lax.rsqrt(var + 1e-5) * gam_ref[...] + bet_ref[...]
    xt_ref[0] = jnp.dot(hn, wg_ref[...], preferred_element_type=jnp.float32)
    sc_ref[0] = jnp.dot(hn, ws_ref[...],
                        preferred_element_type=jnp.float32) + bs_ref[...]


def _stage1(xh, wih2, whh2, bias2, m64, gamma2, beta2, wg2, ws2, bs2):
    full = lambda s: pl.BlockSpec(s, lambda b, i: tuple(0 for _ in s))
    return pl.pallas_call(
        _lstm_body,
        grid=(B, NBH),
        in_specs=[
            pl.BlockSpec((1, T, BMH, 2 * FIN), lambda b, i: (b, 0, i, 0)),
            full((2 * FIN, 8 * H)),
            full((2 * H, 8 * H)),
            full((1, 8 * H)),
            full((2 * H, 2 * H)),
            full((1, 2 * H)),
            full((1, 2 * H)),
            full((2 * H, 2 * G)),
            full((2 * H, 2 * G)),
            full((1, 2 * G)),
        ],
        out_specs=[
            pl.BlockSpec((1, BMH, 2 * G), lambda b, i: (b, i, 0)),
            pl.BlockSpec((1, BMH, 2 * G), lambda b, i: (b, i, 0)),
        ],
        out_shape=[
            jax.ShapeDtypeStruct((B, N // 2, 2 * G), jnp.float32),
            jax.ShapeDtypeStruct((B, N // 2, 2 * G), jnp.float32),
        ],
    )(xh, wih2, whh2, bias2, m64, gamma2, beta2, wg2, ws2, bs2)


def _pack_weights(W_ih, W_hh, b_ih, b_hh, ln_gamma, ln_beta, W_g, W_s, b_s):
    wih_t = W_ih.T
    whh_t = W_hh.T
    bvec = b_ih + b_hh
    wih2 = jnp.zeros((2 * FIN, 8 * H), jnp.float32)
    whh2 = jnp.zeros((2 * H, 8 * H), jnp.float32)
    bias2 = jnp.zeros((8 * H,), jnp.float32)
    for k in range(4):
        wih2 = wih2.at[0:FIN, 2 * k * H:(2 * k + 1) * H].set(
            wih_t[:, k * H:(k + 1) * H])
        wih2 = wih2.at[FIN:2 * FIN, (2 * k + 1) * H:(2 * k + 2) * H].set(
            wih_t[:, k * H:(k + 1) * H])
        whh2 = whh2.at[0:H, 2 * k * H:(2 * k + 1) * H].set(
            whh_t[:, k * H:(k + 1) * H])
        whh2 = whh2.at[H:2 * H, (2 * k + 1) * H:(2 * k + 2) * H].set(
            whh_t[:, k * H:(k + 1) * H])
        bias2 = bias2.at[2 * k * H:(2 * k + 1) * H].set(bvec[k * H:(k + 1) * H])
        bias2 = bias2.at[(2 * k + 1) * H:(2 * k + 2) * H].set(
            bvec[k * H:(k + 1) * H])
    ones64 = jnp.ones((H, H), jnp.float32) / H
    m64 = jnp.zeros((2 * H, 2 * H), jnp.float32)
    m64 = m64.at[0:H, 0:H].set(ones64).at[H:2 * H, H:2 * H].set(ones64)
    gamma2 = jnp.concatenate([ln_gamma, ln_gamma]).reshape(1, 2 * H)
    beta2 = jnp.concatenate([ln_beta, ln_beta]).reshape(1, 2 * H)
    wg2 = jnp.zeros((2 * H, 2 * G), jnp.float32)
    wg2 = wg2.at[0:H, 0:G].set(W_g).at[H:2 * H, G:2 * G].set(W_g)
    ws2 = jnp.zeros((2 * H, 2 * G), jnp.float32)
    ws2 = ws2.at[0:H, 0:G].set(W_s).at[H:2 * H, G:2 * G].set(W_s)
    bs2 = jnp.concatenate([b_s, b_s]).reshape(1, 2 * G)
    return wih2, whh2, bias2.reshape(1, 8 * H), m64, gamma2, beta2, wg2, ws2, bs2


def _repack_body(in_ref, out_ref):
    v = in_ref[...]
    out_ref[...] = jnp.concatenate([v[:, 0, :], v[:, 1, :]], axis=1)


def _repack(xt3):
    return pl.pallas_call(
        _repack_body,
        grid=(B * N // 4 // 1000,),
        in_specs=[pl.BlockSpec((1000, 2, 2 * G), lambda i: (i, 0, 0))],
        out_specs=pl.BlockSpec((1000, 4 * G), lambda i: (i, 0)),
        out_shape=jax.ShapeDtypeStruct((B * N // 4, 4 * G), jnp.float32),
    )(xt3)


NS = 16
SUB = 128
SPC = 4
CH = SUB * SPC
NCHUNK = 98
EPT = CH * NCHUNK
EP = EPT * NS
ZR = 1000
NZ = N // ZR


def _sc_body(xr_hbm, src0_hbm, src1_hbm, dst_hbm, zeros_hbm, out_hbm,
             src_v, dst_v, rows_v, acc,
             sem_i, sem_g0, sem_g1, sem_g2, sem_g3,
             sem_s0, sem_s1, sem_s2, sem_s3):
    c = lax.axis_index("c")
    s = lax.axis_index("s")
    sem_g = [sem_g0, sem_g1, sem_g2, sem_g3]
    sem_s = [sem_s0, sem_s1, sem_s2, sem_s3]
    for k in range(4):
        j = k * NS + s

        @pl.when(j < NZ)
        def _():
            pltpu.sync_copy(zeros_hbm, acc.at[pl.ds(j * ZR, ZR)])

    plsc.subcore_barrier()

    base = s * (NCHUNK * SPC)

    def idx_start(chunk, par):
        row = base + chunk * SPC

        @pl.when(c == 0)
        def _():
            pltpu.make_async_copy(src0_hbm.at[pl.ds(row, SPC)],
                                  src_v.at[par], sem_i).start()

        @pl.when(c == 1)
        def _():
            pltpu.make_async_copy(src1_hbm.at[pl.ds(row, SPC)],
                                  src_v.at[par], sem_i).start()

        pltpu.make_async_copy(dst_hbm.at[pl.ds(row, SPC)],
                              dst_v.at[par], sem_i).start()

    def idx_wait(chunk, par):
        row = base + chunk * SPC
        pltpu.make_async_copy(src0_hbm.at[pl.ds(row, SPC)],
                              src_v.at[par], sem_i).wait()
        pltpu.make_async_copy(dst_hbm.at[pl.ds(row, SPC)],
                              dst_v.at[par], sem_i).wait()

    def gather(par, j, slot):
        return pltpu.make_async_copy(xr_hbm.at[src_v.at[par, j]],
                                     rows_v.at[slot], sem_g[slot])

    def scatter(par, j, slot):
        return pltpu.make_async_copy(rows_v.at[slot],
                                     acc.at[dst_v.at[par, j]], sem_s[slot])

    idx_start(0, 0)
    idx_wait(0, 0)
    gather(0, 0, 0).start()

    def chunk_body(chunk, carry):
        par = lax.rem(chunk, 3)
        parn = lax.rem(chunk + 1, 3)
        parp = lax.rem(chunk + 2, 3)

        @pl.when(chunk < NCHUNK - 1)
        def _():
            idx_start(chunk + 1, parn)

        for j in range(SPC):
            if j < SPC - 1:
                @pl.when(chunk > 0)
                def _():
                    scatter(parp, j + 1, j + 1).wait()

                gather(par, j + 1, j + 1).start()
            else:
                @pl.when(chunk < NCHUNK - 1)
                def _():
                    idx_wait(chunk + 1, parn)
                    scatter(par, 0, 0).wait()
                    gather(parn, 0, 0).start()
            gather(par, j, j).wait()
            scatter(par, j, j).start(add=True)
        return carry

    lax.fori_loop(0, NCHUNK, chunk_body, 0, unroll=False)
    par_last = (NCHUNK - 1) % 3
    for j in range(SPC):
        scatter(par_last, j, j).wait()
    plsc.subcore_barrier()
    for k in range(4):
        j = k * NS + s

        @pl.when(j < NZ)
        def _():
            pltpu.sync_copy(acc.at[pl.ds(j * ZR, ZR)],
                            out_hbm.at[pl.ds(c * N + j * ZR, ZR)])


def _stage2(xr2, src0, src1, dstm, zeros):
    mesh = plsc.VectorSubcoreMesh(core_axis_name="c", subcore_axis_name="s")
    kfn = functools.partial(
        pl.kernel,
        mesh=mesh,
        out_type=jax.ShapeDtypeStruct((B * N, G), jnp.float32),
        scratch_types=[
            pltpu.VMEM((3, SPC, SUB), jnp.int32),
            pltpu.VMEM((3, SPC, SUB), jnp.int32),
            pltpu.VMEM((SPC, SUB, G), jnp.float32),
            pltpu.VMEM_SHARED((N + 8, G), jnp.float32),
            pltpu.SemaphoreType.DMA,
            pltpu.SemaphoreType.DMA,
            pltpu.SemaphoreType.DMA,
            pltpu.SemaphoreType.DMA,
            pltpu.SemaphoreType.DMA,
            pltpu.SemaphoreType.DMA,
            pltpu.SemaphoreType.DMA,
            pltpu.SemaphoreType.DMA,
            pltpu.SemaphoreType.DMA,
        ],
        compiler_params=pltpu.CompilerParams(use_tc_tiling_on_sc=False),
    )(_sc_body)
    return kfn(xr2, src0, src1, dstm, zeros)


BM3 = 2000


def _readout_body(sc_ref, agg_ref, wt_ref, bfc_ref, out_ref):
    v = jnp.maximum(sc_ref[0] + agg_ref[0], 0.0)
    out_ref[0] = jnp.sum(v * wt_ref[...], axis=1, keepdims=True) + bfc_ref[0, 0]


def _stage3(selfc, agg3, wt, bfc):
    full = lambda s: pl.BlockSpec(s, lambda b, i: tuple(0 for _ in s))
    return pl.pallas_call(
        _readout_body,
        grid=(B, N // BM3),
        in_specs=[
            pl.BlockSpec((1, BM3, G), lambda b, i: (b, i, 0)),
            pl.BlockSpec((1, BM3, G), lambda b, i: (b, i, 0)),
            full((1, G)),
            full((1, 1)),
        ],
        out_specs=pl.BlockSpec((1, BM3, 1), lambda b, i: (b, i, 0)),
        out_shape=jax.ShapeDtypeStruct((B, N, 1), jnp.float32),
    )(selfc, agg3, wt, bfc)


def kernel(x, W_ih, W_hh, b_ih, b_hh, ln_gamma, ln_beta, W_g, W_s, b_s,
           W_fc, b_fc, adj_val, src, dst):
    packed = _pack_weights(W_ih, W_hh, b_ih, b_hh, ln_gamma, ln_beta,
                           W_g, W_s, b_s)
    xh = x.reshape(B, T, N // 2, 2 * FIN)
    xt_all, selfc_p = _stage1(xh, *packed)
    selfc = selfc_p.reshape(B, N, G)
    xt4 = _repack(xt_all.reshape(B * N // 4, 2, 2 * G))
    xr2 = xt4.reshape(B * N, G)

    pad = EP - E
    src_p = jnp.concatenate([src, jnp.zeros((pad,), jnp.int32)])
    dst_p = jnp.concatenate([dst, jnp.full((pad,), N, jnp.int32)])
    src0 = src_p.reshape(EP // SUB, SUB)
    src1 = src0 + N
    dstm = dst_p.reshape(EP // SUB, SUB)
    zeros = jnp.zeros((ZR, G), jnp.float32)

    agg2 = _stage2(xr2, src0, src1, dstm, zeros)
    agg3 = agg2.reshape(B, N, G)

    return _stage3(selfc, agg3, W_fc.T.reshape(1, G), b_fc.reshape(1, 1))


# speedup vs baseline: 6.5668x; 1.1269x over previous
import functools

import jax
import jax.numpy as jnp
from jax import lax
from jax.experimental import pallas as pl
from jax.experimental.pallas import tpu as pltpu
from jax.experimental.pallas import tpu_sc as plsc

N = 50000
E = 800000
B = 2
T = 16
FIN = 8
H = 64
G = 32


BMH = 1000
NBH = (N // 2) // BMH


def _sigt(v):
    return 0.5 * jnp.tanh(0.5 * v) + 0.5


def _lstm_body(x_ref, wih_ref, whh_ref, bias_ref, m64_ref, gam_ref, bet_ref,
               wg_ref, ws_ref, bs_ref, xt_ref, sc_ref):
    xb = x_ref[0]
    hp = jnp.zeros((BMH, 2 * H), jnp.float32)
    cp = jnp.zeros((BMH, 2 * H), jnp.float32)
    whh = whh_ref[...]
    bias = bias_ref[...]
    for t in range(T):
        g_all = (jnp.dot(xb, wih_ref[t], preferred_element_type=jnp.float32)
                 + jnp.dot(hp, whh, preferred_element_type=jnp.float32)
                 + bias)
        gi = _sigt(g_all[:, 0:2 * H])
        gf = _sigt(g_all[:, 2 * H:4 * H])
        gg = jnp.tanh(g_all[:, 4 * H:6 * H])
        go = _sigt(g_all[:, 6 * H:8 * H])
        cp = gf * cp + gi * gg
        hp = go * jnp.tanh(cp)
    mu = jnp.dot(hp, m64_ref[...], preferred_element_type=jnp.float32)
    d = hp - mu
    var = jnp.dot(d * d, m64_ref[...], preferred_element_type=jnp.float32)
    hn = d * lax.rsqrt(var + 1e-5) * gam_ref[...] + bet_ref[...]
    xt_ref[0] = jnp.dot(hn, wg_ref[...], preferred_element_type=jnp.float32)
    sc_ref[0] = jnp.dot(hn, ws_ref[...],
                        preferred_element_type=jnp.float32) + bs_ref[...]


def _stage1(xh, wih2, whh2, bias2, m64, gamma2, beta2, wg2, ws2, bs2):
    full = lambda s: pl.BlockSpec(s, lambda b, i: tuple(0 for _ in s))
    return pl.pallas_call(
        _lstm_body,
        grid=(B, NBH),
        in_specs=[
            pl.BlockSpec((1, BMH, T * 2 * FIN), lambda b, i: (b, i, 0)),
            full((T, T * 2 * FIN, 8 * H)),
            full((2 * H, 8 * H)),
            full((1, 8 * H)),
            full((2 * H, 2 * H)),
            full((1, 2 * H)),
            full((1, 2 * H)),
            full((2 * H, 2 * G)),
            full((2 * H, 2 * G)),
            full((1, 2 * G)),
        ],
        out_specs=[
            pl.BlockSpec((1, BMH, 2 * G), lambda b, i: (b, i, 0)),
            pl.BlockSpec((1, BMH, 2 * G), lambda b, i: (b, i, 0)),
        ],
        out_shape=[
            jax.ShapeDtypeStruct((B, N // 2, 2 * G), jnp.float32),
            jax.ShapeDtypeStruct((B, N // 2, 2 * G), jnp.float32),
        ],
    )(xh, wih2, whh2, bias2, m64, gamma2, beta2, wg2, ws2, bs2)


def _pack_weights(W_ih, W_hh, b_ih, b_hh, ln_gamma, ln_beta, W_g, W_s, b_s):
    wih_t = W_ih.T
    whh_t = W_hh.T
    bvec = b_ih + b_hh
    wih2 = jnp.zeros((2 * FIN, 8 * H), jnp.float32)
    whh2 = jnp.zeros((2 * H, 8 * H), jnp.float32)
    bias2 = jnp.zeros((8 * H,), jnp.float32)
    for k in range(4):
        wih2 = wih2.at[0:FIN, 2 * k * H:(2 * k + 1) * H].set(
            wih_t[:, k * H:(k + 1) * H])
        wih2 = wih2.at[FIN:2 * FIN, (2 * k + 1) * H:(2 * k + 2) * H].set(
            wih_t[:, k * H:(k + 1) * H])
        whh2 = whh2.at[0:H, 2 * k * H:(2 * k + 1) * H].set(
            whh_t[:, k * H:(k + 1) * H])
        whh2 = whh2.at[H:2 * H, (2 * k + 1) * H:(2 * k + 2) * H].set(
            whh_t[:, k * H:(k + 1) * H])
        bias2 = bias2.at[2 * k * H:(2 * k + 1) * H].set(bvec[k * H:(k + 1) * H])
        bias2 = bias2.at[(2 * k + 1) * H:(2 * k + 2) * H].set(
            bvec[k * H:(k + 1) * H])
    wih_big = jnp.zeros((T, T * 2 * FIN, 8 * H), jnp.float32)
    for t in range(T):
        wih_big = wih_big.at[t, t * 2 * FIN:(t + 1) * 2 * FIN, :].set(wih2)
    ones64 = jnp.ones((H, H), jnp.float32) / H
    m64 = jnp.zeros((2 * H, 2 * H), jnp.float32)
    m64 = m64.at[0:H, 0:H].set(ones64).at[H:2 * H, H:2 * H].set(ones64)
    gamma2 = jnp.concatenate([ln_gamma, ln_gamma]).reshape(1, 2 * H)
    beta2 = jnp.concatenate([ln_beta, ln_beta]).reshape(1, 2 * H)
    wg2 = jnp.zeros((2 * H, 2 * G), jnp.float32)
    wg2 = wg2.at[0:H, 0:G].set(W_g).at[H:2 * H, G:2 * G].set(W_g)
    ws2 = jnp.zeros((2 * H, 2 * G), jnp.float32)
    ws2 = ws2.at[0:H, 0:G].set(W_s).at[H:2 * H, G:2 * G].set(W_s)
    bs2 = jnp.concatenate([b_s, b_s]).reshape(1, 2 * G)
    return (wih_big, whh2, bias2.reshape(1, 8 * H), m64, gamma2, beta2,
            wg2, ws2, bs2)


def _repack_body(in_ref, out_ref):
    v = in_ref[...]
    out_ref[...] = jnp.concatenate([v[:, 0, :], v[:, 1, :]], axis=1)


def _repack(xt3):
    return pl.pallas_call(
        _repack_body,
        grid=(B * N // 4 // 1000,),
        in_specs=[pl.BlockSpec((1000, 2, 2 * G), lambda i: (i, 0, 0))],
        out_specs=pl.BlockSpec((1000, 4 * G), lambda i: (i, 0)),
        out_shape=jax.ShapeDtypeStruct((B * N // 4, 4 * G), jnp.float32),
    )(xt3)


NS = 16
SUB = 128
SPC = 4
CH = SUB * SPC
NCHUNK = 98
EPT = CH * NCHUNK
EP = EPT * NS
ZR = 1000
NZ = N // ZR


def _sc_body(xr_hbm, src0_hbm, src1_hbm, dst_hbm, zeros_hbm, out_hbm,
             src_v, dst_v, rows_v, acc,
             sem_i, sem_g0, sem_g1, sem_g2, sem_g3,
             sem_s0, sem_s1, sem_s2, sem_s3):
    c = lax.axis_index("c")
    s = lax.axis_index("s")
    sem_g = [sem_g0, sem_g1, sem_g2, sem_g3]
    sem_s = [sem_s0, sem_s1, sem_s2, sem_s3]
    for k in range(4):
        j = k * NS + s

        @pl.when(j < NZ)
        def _():
            pltpu.sync_copy(zeros_hbm, acc.at[pl.ds(j * ZR, ZR)])

    plsc.subcore_barrier()

    base = s * (NCHUNK * SPC)

    def idx_start(chunk, par):
        row = base + chunk * SPC

        @pl.when(c == 0)
        def _():
            pltpu.make_async_copy(src0_hbm.at[pl.ds(row, SPC)],
                                  src_v.at[par], sem_i).start()

        @pl.when(c == 1)
        def _():
            pltpu.make_async_copy(src1_hbm.at[pl.ds(row, SPC)],
                                  src_v.at[par], sem_i).start()

        pltpu.make_async_copy(dst_hbm.at[pl.ds(row, SPC)],
                              dst_v.at[par], sem_i).start()

    def idx_wait(chunk, par):
        row = base + chunk * SPC
        pltpu.make_async_copy(src0_hbm.at[pl.ds(row, SPC)],
                              src_v.at[par], sem_i).wait()
        pltpu.make_async_copy(dst_hbm.at[pl.ds(row, SPC)],
                              dst_v.at[par], sem_i).wait()

    def gather(par, j, slot):
        return pltpu.make_async_copy(xr_hbm.at[src_v.at[par, j]],
                                     rows_v.at[slot], sem_g[slot])

    def scatter(par, j, slot):
        return pltpu.make_async_copy(rows_v.at[slot],
                                     acc.at[dst_v.at[par, j]], sem_s[slot])

    idx_start(0, 0)
    idx_wait(0, 0)
    gather(0, 0, 0).start()

    def chunk_body(chunk, carry):
        par = lax.rem(chunk, 3)
        parn = lax.rem(chunk + 1, 3)
        parp = lax.rem(chunk + 2, 3)

        @pl.when(chunk < NCHUNK - 1)
        def _():
            idx_start(chunk + 1, parn)

        for j in range(SPC):
            if j < SPC - 1:
                @pl.when(chunk > 0)
                def _():
                    scatter(parp, j + 1, j + 1).wait()

                gather(par, j + 1, j + 1).start()
            else:
                @pl.when(chunk < NCHUNK - 1)
                def _():
                    idx_wait(chunk + 1, parn)
                    scatter(par, 0, 0).wait()
                    gather(parn, 0, 0).start()
            gather(par, j, j).wait()
            scatter(par, j, j).start(add=True)
        return carry

    lax.fori_loop(0, NCHUNK, chunk_body, 0, unroll=False)
    par_last = (NCHUNK - 1) % 3
    for j in range(SPC):
        scatter(par_last, j, j).wait()
    plsc.subcore_barrier()
    for k in range(4):
        j = k * NS + s

        @pl.when(j < NZ)
        def _():
            pltpu.sync_copy(acc.at[pl.ds(j * ZR, ZR)],
                            out_hbm.at[pl.ds(c * N + j * ZR, ZR)])


def _stage2(xr2, src0, src1, dstm, zeros):
    mesh = plsc.VectorSubcoreMesh(core_axis_name="c", subcore_axis_name="s")
    kfn = functools.partial(
        pl.kernel,
        mesh=mesh,
        out_type=jax.ShapeDtypeStruct((B * N, G), jnp.float32),
        scratch_types=[
            pltpu.VMEM((3, SPC, SUB), jnp.int32),
            pltpu.VMEM((3, SPC, SUB), jnp.int32),
            pltpu.VMEM((SPC, SUB, G), jnp.float32),
            pltpu.VMEM_SHARED((N + 8, G), jnp.float32),
            pltpu.SemaphoreType.DMA,
            pltpu.SemaphoreType.DMA,
            pltpu.SemaphoreType.DMA,
            pltpu.SemaphoreType.DMA,
            pltpu.SemaphoreType.DMA,
            pltpu.SemaphoreType.DMA,
            pltpu.SemaphoreType.DMA,
            pltpu.SemaphoreType.DMA,
            pltpu.SemaphoreType.DMA,
        ],
        compiler_params=pltpu.CompilerParams(use_tc_tiling_on_sc=False),
    )(_sc_body)
    return kfn(xr2, src0, src1, dstm, zeros)


BM3 = 2000


def _readout_body(sc_ref, agg_ref, wt_ref, bfc_ref, out_ref):
    v = jnp.maximum(sc_ref[0] + agg_ref[0], 0.0)
    out_ref[0] = jnp.sum(v * wt_ref[...], axis=1, keepdims=True) + bfc_ref[0, 0]


def _stage3(selfc, agg3, wt, bfc):
    full = lambda s: pl.BlockSpec(s, lambda b, i: tuple(0 for _ in s))
    return pl.pallas_call(
        _readout_body,
        grid=(B, N // BM3),
        in_specs=[
            pl.BlockSpec((1, BM3, G), lambda b, i: (b, i, 0)),
            pl.BlockSpec((1, BM3, G), lambda b, i: (b, i, 0)),
            full((1, G)),
            full((1, 1)),
        ],
        out_specs=pl.BlockSpec((1, BM3, 1), lambda b, i: (b, i, 0)),
        out_shape=jax.ShapeDtypeStruct((B, N, 1), jnp.float32),
    )(selfc, agg3, wt, bfc)


def kernel(x, W_ih, W_hh, b_ih, b_hh, ln_gamma, ln_beta, W_g, W_s, b_s,
           W_fc, b_fc, adj_val, src, dst):
    packed = _pack_weights(W_ih, W_hh, b_ih, b_hh, ln_gamma, ln_beta,
                           W_g, W_s, b_s)
    xh = jnp.transpose(x.reshape(B, T, N // 2, 2 * FIN),
                       (0, 2, 1, 3)).reshape(B, N // 2, T * 2 * FIN)
    xt_all, selfc_p = _stage1(xh, *packed)
    selfc = selfc_p.reshape(B, N, G)
    xt4 = _repack(xt_all.reshape(B * N // 4, 2, 2 * G))
    xr2 = xt4.reshape(B * N, G)

    pad = EP - E
    src_p = jnp.concatenate([src, jnp.zeros((pad,), jnp.int32)])
    dst_p = jnp.concatenate([dst, jnp.full((pad,), N, jnp.int32)])
    src0 = src_p.reshape(EP // SUB, SUB)
    src1 = src0 + N
    dstm = dst_p.reshape(EP // SUB, SUB)
    zeros = jnp.zeros((ZR, G), jnp.float32)

    agg2 = _stage2(xr2, src0, src1, dstm, zeros)
    agg3 = agg2.reshape(B, N, G)

    return _stage3(selfc, agg3, W_fc.T.reshape(1, G), b_fc.reshape(1, 1))
